# Initial kernel scaffold; baseline (speedup 1.0000x reference)
import jax
import jax.numpy as jnp
from jax.experimental import pallas as pl


def kernel(x, emb_table, W, b):
    raise NotImplementedError("write your pallas kernel here")


# speedup vs baseline: 168.5474x; 168.5474x over previous
import functools

import jax
import jax.numpy as jnp
from jax import lax
from jax.experimental import pallas as pl
from jax.experimental.pallas import tpu as pltpu
from jax.experimental.pallas import tpu_sc as plsc

BATCH = 16384
HIST = 200
VOCAB = 1500
EMB_DIM = 4

L = 16
NC, NS = 2, 16
NW = NC * NS
ROWS_PER_W = BATCH // NW
XW_PER_W = ROWS_PER_W * HIST
VPAD = 1504
EPAD = VPAD * EMB_DIM


def _body(x_hbm, emb_hbm, wb_hbm, out_hbm, x_v, emb_v, s_v, wb_v, out_v):
    wid = lax.axis_index("s") * NC + lax.axis_index("c")
    iota = lax.iota(jnp.int32, L)

    pltpu.sync_copy(wb_hbm, wb_v)
    pltpu.sync_copy(emb_hbm, emb_v.at[pl.ds(0, VOCAB * EMB_DIM)])

    def bcast(i):
        return plsc.load_gather(wb_v, [jnp.full((L,), i, jnp.int32)])

    w0, w1, w2, w3, bb = bcast(1), bcast(2), bcast(3), bcast(4), bcast(5)

    def s_step(g, carry):
        fidx = (g * L + iota) * EMB_DIM
        e0 = plsc.load_gather(emb_v, [fidx])
        e1 = plsc.load_gather(emb_v, [fidx + 1])
        e2 = plsc.load_gather(emb_v, [fidx + 2])
        e3 = plsc.load_gather(emb_v, [fidx + 3])
        y = e0 * w0 + e1 * w1 + e2 * w2 + e3 * w3 + bb
        s_v[pl.ds(g * L, L)] = jnp.maximum(y, 0.0)
        return carry

    lax.fori_loop(0, VPAD // L, s_step, 0)

    base = wid * ROWS_PER_W
    pltpu.sync_copy(x_hbm.at[pl.ds(base * HIST, XW_PER_W)], x_v)

    def g_step(g, carry):
        roff = (g * L + iota) * HIST

        def j_step(j, acc):
            xi = plsc.load_gather(x_v, [roff + j])
            return acc + plsc.load_gather(s_v, [xi])

        acc = lax.fori_loop(0, HIST, j_step, jnp.zeros((L,), jnp.float32),
                            unroll=8)
        out_v[pl.ds(g * L, L)] = acc
        return carry

    lax.fori_loop(0, ROWS_PER_W // L, g_step, 0)
    pltpu.sync_copy(out_v, out_hbm.at[pl.ds(base, ROWS_PER_W)])


_sc_call = functools.partial(
    pl.kernel,
    mesh=plsc.VectorSubcoreMesh(core_axis_name="c", subcore_axis_name="s"),
    out_type=jax.ShapeDtypeStruct((BATCH,), jnp.float32),
    compiler_params=pltpu.CompilerParams(needs_layout_passes=False),
    scratch_types=[
        pltpu.VMEM((XW_PER_W,), jnp.int32),
        pltpu.VMEM((EPAD,), jnp.float32),
        pltpu.VMEM((VPAD,), jnp.float32),
        pltpu.VMEM((L,), jnp.float32),
        pltpu.VMEM((ROWS_PER_W,), jnp.float32),
    ],
)(_body)


def kernel(x, emb_table, W, b):
    x_flat = x.reshape(-1).astype(jnp.int32)
    emb_flat = emb_table.reshape(-1)
    wb = jnp.zeros((L,), jnp.float32).at[1:1 + EMB_DIM].set(W[0]).at[1 + EMB_DIM].set(b[0])
    return _sc_call(x_flat, emb_flat, wb)


# speedup vs baseline: 185.5619x; 1.1009x over previous
import functools

import jax
import jax.numpy as jnp
from jax import lax
from jax.experimental import pallas as pl
from jax.experimental.pallas import tpu as pltpu
from jax.experimental.pallas import tpu_sc as plsc

BATCH = 16384
HIST = 200
VOCAB = 1500
EMB_DIM = 4

L = 16
NC, NS = 2, 16
NW = NC * NS
ROWS_PER_W = BATCH // NW
NCHUNK = 4
CROWS = ROWS_PER_W // NCHUNK
CWORDS = CROWS * HIST
VPAD = 1504
EPAD = VPAD * EMB_DIM


def _body(x_hbm, emb_hbm, wb_hbm, out_hbm,
          xb0, xb1, emb_v, s_v, wb_v, out_v, sem0, sem1):
    wid = lax.axis_index("s") * NC + lax.axis_index("c")
    iota = lax.iota(jnp.int32, L)
    xbufs = (xb0, xb1)
    sems = (sem0, sem1)
    xbase = wid * (ROWS_PER_W * HIST)

    def chunk_copy(c):
        return pltpu.make_async_copy(
            x_hbm.at[pl.ds(xbase + c * CWORDS, CWORDS)],
            xbufs[c % 2], sems[c % 2])

    chunk_copy(0).start()

    pltpu.sync_copy(wb_hbm, wb_v)
    pltpu.sync_copy(emb_hbm, emb_v.at[pl.ds(0, VOCAB * EMB_DIM)])

    def bcast(i):
        return plsc.load_gather(wb_v, [jnp.full((L,), i, jnp.int32)])

    w0, w1, w2, w3, bb = bcast(1), bcast(2), bcast(3), bcast(4), bcast(5)

    def s_step(g, carry):
        fidx = (g * L + iota) * EMB_DIM
        e0 = plsc.load_gather(emb_v, [fidx])
        e1 = plsc.load_gather(emb_v, [fidx + 1])
        e2 = plsc.load_gather(emb_v, [fidx + 2])
        e3 = plsc.load_gather(emb_v, [fidx + 3])
        y = e0 * w0 + e1 * w1 + e2 * w2 + e3 * w3 + bb
        s_v[pl.ds(g * L, L)] = jnp.maximum(y, 0.0)
        return carry

    lax.fori_loop(0, VPAD // L, s_step, 0)

    zero4 = (jnp.zeros((L,), jnp.float32),) * 4

    for c in range(NCHUNK):
        chunk_copy(c).wait()
        if c + 1 < NCHUNK:
            chunk_copy(c + 1).start()
        x_v = xbufs[c % 2]

        def g_step(g, carry):
            base = (g * L + iota) * HIST + iota

            def j_step(jj, accs):
                a = list(accs)
                for k in range(4):
                    j = jj * 4 + k
                    addr = base + j
                    addr = jnp.where(iota >= HIST - j, addr - HIST, addr)
                    xi = plsc.load_gather(x_v, [addr])
                    a[k] = a[k] + plsc.load_gather(s_v, [xi])
                return tuple(a)

            a0, a1, a2, a3 = lax.fori_loop(0, HIST // 4, j_step, zero4,
                                           unroll=2)
            out_v[pl.ds(c * CROWS + g * L, L)] = (a0 + a1) + (a2 + a3)
            return carry

        lax.fori_loop(0, CROWS // L, g_step, 0)

    pltpu.sync_copy(out_v, out_hbm.at[pl.ds(wid * ROWS_PER_W, ROWS_PER_W)])


_sc_call = functools.partial(
    pl.kernel,
    mesh=plsc.VectorSubcoreMesh(core_axis_name="c", subcore_axis_name="s"),
    out_type=jax.ShapeDtypeStruct((BATCH,), jnp.float32),
    compiler_params=pltpu.CompilerParams(needs_layout_passes=False),
    scratch_types=[
        pltpu.VMEM((CWORDS,), jnp.int32),
        pltpu.VMEM((CWORDS,), jnp.int32),
        pltpu.VMEM((EPAD,), jnp.float32),
        pltpu.VMEM((VPAD,), jnp.float32),
        pltpu.VMEM((L,), jnp.float32),
        pltpu.VMEM((ROWS_PER_W,), jnp.float32),
        pltpu.SemaphoreType.DMA,
        pltpu.SemaphoreType.DMA,
    ],
)(_body)


def kernel(x, emb_table, W, b):
    x_flat = x.reshape(-1).astype(jnp.int32)
    emb_flat = emb_table.reshape(-1)
    wb = jnp.zeros((L,), jnp.float32).at[1:1 + EMB_DIM].set(W[0]).at[1 + EMB_DIM].set(b[0])
    return _sc_call(x_flat, emb_flat, wb)


# speedup vs baseline: 243.5358x; 1.3124x over previous
import functools

import jax
import jax.numpy as jnp
from jax import lax
from jax.experimental import pallas as pl
from jax.experimental.pallas import tpu as pltpu
from jax.experimental.pallas import tpu_sc as plsc

BATCH = 16384
HIST = 200
VOCAB = 1500
EMB_DIM = 4

L = 16
NC, NS = 2, 16
NW = NC * NS
ROWS_PER_W = BATCH // NW
NCHUNK = 4
CROWS = ROWS_PER_W // NCHUNK
VPAD = 1504
EPAD = VPAD * EMB_DIM


def _body(x_hbm, emb_hbm, wb_hbm, out_hbm,
          xb0, xb1, emb_v, s_v, wb_v, out_v, sem0, sem1):
    wid = lax.axis_index("s") * NC + lax.axis_index("c")
    iota = lax.iota(jnp.int32, L)
    xbufs = (xb0, xb1)
    sems = (sem0, sem1)
    row_base = wid * ROWS_PER_W

    def chunk_copy(c):
        return pltpu.make_async_copy(
            x_hbm.at[pl.ds(row_base + c * CROWS, CROWS), :],
            xbufs[c % 2], sems[c % 2])

    chunk_copy(0).start()

    pltpu.sync_copy(wb_hbm, wb_v)
    pltpu.sync_copy(emb_hbm, emb_v.at[pl.ds(0, VOCAB * EMB_DIM)])

    def bcast(i):
        return plsc.load_gather(wb_v, [jnp.full((L,), i, jnp.int32)])

    w0, w1, w2, w3, bb = bcast(1), bcast(2), bcast(3), bcast(4), bcast(5)

    def s_step(g, carry):
        fidx = (g * L + iota) * EMB_DIM
        e0 = plsc.load_gather(emb_v, [fidx])
        e1 = plsc.load_gather(emb_v, [fidx + 1])
        e2 = plsc.load_gather(emb_v, [fidx + 2])
        e3 = plsc.load_gather(emb_v, [fidx + 3])
        y = e0 * w0 + e1 * w1 + e2 * w2 + e3 * w3 + bb
        s_v[pl.ds(g * L, L)] = jnp.maximum(y, 0.0)
        return carry

    lax.fori_loop(0, VPAD // L, s_step, 0)

    zero4 = (jnp.zeros((L,), jnp.float32),) * 4

    for c in range(NCHUNK):
        chunk_copy(c).wait()
        if c + 1 < NCHUNK:
            chunk_copy(c + 1).start()
        x_v = xbufs[c % 2]

        def g_step(g, carry):
            rows = g * L + iota

            def j_step(jj, accs):
                a = list(accs)
                for k in range(4):
                    j = jj * 4 + k
                    col = iota + j
                    col = jnp.where(col >= HIST, col - HIST, col)
                    xi = plsc.load_gather(x_v, [rows, col])
                    a[k] = a[k] + plsc.load_gather(s_v, [xi])
                return tuple(a)

            a0, a1, a2, a3 = lax.fori_loop(0, HIST // 4, j_step, zero4,
                                           unroll=2)
            out_v[pl.ds(c * CROWS + g * L, L)] = (a0 + a1) + (a2 + a3)
            return carry

        lax.fori_loop(0, CROWS // L, g_step, 0)

    pltpu.sync_copy(out_v, out_hbm.at[pl.ds(row_base, ROWS_PER_W)])


_sc_call = functools.partial(
    pl.kernel,
    mesh=plsc.VectorSubcoreMesh(core_axis_name="c", subcore_axis_name="s"),
    out_type=jax.ShapeDtypeStruct((BATCH,), jnp.float32),
    compiler_params=pltpu.CompilerParams(needs_layout_passes=False,
                                         use_tc_tiling_on_sc=True),
    scratch_types=[
        pltpu.VMEM((CROWS, HIST), jnp.int32),
        pltpu.VMEM((CROWS, HIST), jnp.int32),
        pltpu.VMEM((EPAD,), jnp.float32),
        pltpu.VMEM((VPAD,), jnp.float32),
        pltpu.VMEM((L,), jnp.float32),
        pltpu.VMEM((ROWS_PER_W,), jnp.float32),
        pltpu.SemaphoreType.DMA,
        pltpu.SemaphoreType.DMA,
    ],
)(_body)


def kernel(x, emb_table, W, b):
    emb_flat = emb_table.reshape(-1)
    wb = jnp.zeros((L,), jnp.float32).at[1:1 + EMB_DIM].set(W[0]).at[1 + EMB_DIM].set(b[0])
    return _sc_call(x.astype(jnp.int32), emb_flat, wb)


# speedup vs baseline: 375.6369x; 1.5424x over previous
import functools

import jax
import jax.numpy as jnp
from jax import lax
from jax.experimental import pallas as pl
from jax.experimental.pallas import tpu as pltpu
from jax.experimental.pallas import tpu_sc as plsc

BATCH = 16384
HIST = 200
VOCAB = 1500
EMB_DIM = 4

L = 16
NC, NS = 2, 16
NW = NC * NS
ROWS_PER_W = BATCH // NW
NCHUNK = 4
CROWS = ROWS_PER_W // NCHUNK
VPAD = 1504
EPAD = VPAD * EMB_DIM


def _body(xt_hbm, emb_hbm, wb_hbm, out_hbm,
          xb0, xb1, emb_v, s_v, wb_v, out_v, sem0, sem1):
    wid = lax.axis_index("s") * NC + lax.axis_index("c")
    iota = lax.iota(jnp.int32, L)
    xbufs = (xb0, xb1)
    sems = (sem0, sem1)
    col_base = wid * ROWS_PER_W

    def chunk_copy(c):
        return pltpu.make_async_copy(
            xt_hbm.at[:, pl.ds(col_base + c * CROWS, CROWS)],
            xbufs[c % 2], sems[c % 2])

    chunk_copy(0).start()

    pltpu.sync_copy(wb_hbm, wb_v)
    pltpu.sync_copy(emb_hbm, emb_v.at[pl.ds(0, VOCAB * EMB_DIM)])

    def bcast(i):
        return plsc.load_gather(wb_v, [jnp.full((L,), i, jnp.int32)])

    w0, w1, w2, w3, bb = bcast(1), bcast(2), bcast(3), bcast(4), bcast(5)

    def s_step(g, carry):
        fidx = (g * L + iota) * EMB_DIM
        e0 = plsc.load_gather(emb_v, [fidx])
        e1 = plsc.load_gather(emb_v, [fidx + 1])
        e2 = plsc.load_gather(emb_v, [fidx + 2])
        e3 = plsc.load_gather(emb_v, [fidx + 3])
        y = e0 * w0 + e1 * w1 + e2 * w2 + e3 * w3 + bb
        s_v[pl.ds(g * L, L)] = jnp.maximum(y, 0.0)
        return carry

    lax.fori_loop(0, VPAD // L, s_step, 0)

    zero4 = (jnp.zeros((L,), jnp.float32),) * 4

    for c in range(NCHUNK):
        chunk_copy(c).wait()
        if c + 1 < NCHUNK:
            chunk_copy(c + 1).start()
        x_v = xbufs[c % 2]

        def g_step(g, carry):
            gb = g * L

            def j_step(jj, accs):
                a = list(accs)
                for k in range(4):
                    j = jj * 4 + k
                    xi = x_v[j, pl.ds(gb, L)]
                    a[k] = a[k] + plsc.load_gather(s_v, [xi])
                return tuple(a)

            a0, a1, a2, a3 = lax.fori_loop(0, HIST // 4, j_step, zero4,
                                           unroll=2)
            out_v[pl.ds(c * CROWS + gb, L)] = (a0 + a1) + (a2 + a3)
            return carry

        lax.fori_loop(0, CROWS // L, g_step, 0)

    pltpu.sync_copy(out_v, out_hbm.at[pl.ds(col_base, ROWS_PER_W)])


_sc_call = functools.partial(
    pl.kernel,
    mesh=plsc.VectorSubcoreMesh(core_axis_name="c", subcore_axis_name="s"),
    out_type=jax.ShapeDtypeStruct((BATCH,), jnp.float32),
    compiler_params=pltpu.CompilerParams(needs_layout_passes=False,
                                         use_tc_tiling_on_sc=True),
    scratch_types=[
        pltpu.VMEM((HIST, CROWS), jnp.int32),
        pltpu.VMEM((HIST, CROWS), jnp.int32),
        pltpu.VMEM((EPAD,), jnp.float32),
        pltpu.VMEM((VPAD,), jnp.float32),
        pltpu.VMEM((L,), jnp.float32),
        pltpu.VMEM((ROWS_PER_W,), jnp.float32),
        pltpu.SemaphoreType.DMA,
        pltpu.SemaphoreType.DMA,
    ],
)(_body)


def kernel(x, emb_table, W, b):
    emb_flat = emb_table.reshape(-1)
    wb = jnp.zeros((L,), jnp.float32).at[1:1 + EMB_DIM].set(W[0]).at[1 + EMB_DIM].set(b[0])
    return _sc_call(x.T.astype(jnp.int32), emb_flat, wb)
